# Initial kernel scaffold; baseline (speedup 1.0000x reference)
import jax
import jax.numpy as jnp
from jax.experimental import pallas as pl


def kernel(x, edge_index, W1, b1, W2, b2, Wc1, bc1, Wc2, bc2):
    raise NotImplementedError("write your pallas kernel here")


# speedup vs baseline: 21.4307x; 21.4307x over previous
import jax
import jax.numpy as jnp
from jax import lax
from jax.experimental import pallas as pl
from jax.experimental.pallas import tpu as pltpu
from jax.experimental.pallas import tpu_sc as plsc

N = 10000
D_IN = 128
D_HID = 64
E = 320000

C = 128
NW = 32
CHUNKS = (E + NW * C - 1) // (NW * C)
EW = CHUNKS * C
EPAD = EW * NW
ROWS = 10240
TR = ROWS // 16

_mesh = plsc.VectorSubcoreMesh(core_axis_name="c", subcore_axis_name="s")


def _sc_deg_body(dst_hbm, zeros_hbm, ones_hbm, deg_out, dstv, ones_v, deg_sh):
    c = lax.axis_index("c")
    s = lax.axis_index("s")
    w = s * 2 + c
    pltpu.sync_copy(zeros_hbm, deg_sh.at[pl.ds(s * TR, TR)])
    pltpu.sync_copy(dst_hbm.at[w], dstv)
    pltpu.sync_copy(ones_hbm, ones_v)
    plsc.subcore_barrier()

    def body(j, carry):
        pltpu.sync_copy(ones_v, deg_sh.at[dstv.at[j]], add=True)
        return carry

    lax.fori_loop(0, CHUNKS, body, 0)
    plsc.subcore_barrier()
    pltpu.sync_copy(deg_sh.at[pl.ds(s * TR, TR)],
                    deg_out.at[c, pl.ds(s * TR, TR)])


_sc_deg = pl.kernel(
    _sc_deg_body,
    mesh=_mesh,
    out_type=jax.ShapeDtypeStruct((2, ROWS, 1), jnp.float32),
    scratch_types=[
        pltpu.VMEM((CHUNKS, C), jnp.int32),
        pltpu.VMEM((C, 1), jnp.float32),
        pltpu.VMEM_SHARED((ROWS, 1), jnp.float32),
    ],
    compiler_params=pltpu.CompilerParams(use_tc_tiling_on_sc=False),
)


def _sc_agg_body(g_hbm, src_hbm, dst_hbm, zeros_hbm, acc_out,
                 srcv, dstv, rows_v, acc_sh, sem):
    c = lax.axis_index("c")
    s = lax.axis_index("s")
    w = s * 2 + c
    pltpu.sync_copy(zeros_hbm, acc_sh.at[pl.ds(s * TR, TR)])
    pltpu.sync_copy(src_hbm.at[w], srcv)
    pltpu.sync_copy(dst_hbm.at[w], dstv)
    plsc.subcore_barrier()

    def body(j, carry):
        pltpu.async_copy(g_hbm.at[srcv.at[j]], rows_v, sem).wait()
        pltpu.sync_copy(rows_v, acc_sh.at[dstv.at[j]], add=True)
        return carry

    lax.fori_loop(0, CHUNKS, body, 0)
    plsc.subcore_barrier()
    pltpu.sync_copy(acc_sh.at[pl.ds(s * TR, TR)],
                    acc_out.at[c, pl.ds(s * TR, TR)])


_sc_agg = pl.kernel(
    _sc_agg_body,
    mesh=_mesh,
    out_type=jax.ShapeDtypeStruct((2, ROWS, D_HID), jnp.float32),
    scratch_types=[
        pltpu.VMEM((CHUNKS, C), jnp.int32),
        pltpu.VMEM((CHUNKS, C), jnp.int32),
        pltpu.VMEM((C, D_HID), jnp.float32),
        pltpu.VMEM_SHARED((ROWS, D_HID), jnp.float32),
        pltpu.SemaphoreType.DMA,
    ],
    compiler_params=pltpu.CompilerParams(use_tc_tiling_on_sc=False),
)


def _tc_pre_body(x_ref, w1_ref, degp_ref, g1_ref, dis_ref):
    deg = degp_ref[0, :N, :] + degp_ref[1, :N, :] + 1.0
    dis = lax.rsqrt(deg)
    h = jnp.dot(x_ref[...], w1_ref[...], preferred_element_type=jnp.float32)
    g1_ref[...] = h * dis
    dis_ref[...] = dis


def _tc_mid_body(acc_ref, g1_ref, dis_ref, b1_ref, w2_ref, g2_ref):
    a = acc_ref[0, :N, :] + acc_ref[1, :N, :]
    dis = dis_ref[...]
    h1 = jnp.maximum(dis * (a + g1_ref[...]) + b1_ref[...], 0.0)
    h = jnp.dot(h1, w2_ref[...], preferred_element_type=jnp.float32)
    g2_ref[...] = h * dis


def _tc_fin_body(acc_ref, g2_ref, dis_ref, b2_ref, wc1_ref, bc1_ref,
                 wc2_ref, bc2_ref, out_ref):
    a = acc_ref[0, :N, :] + acc_ref[1, :N, :]
    h2 = jnp.maximum(dis_ref[...] * (a + g2_ref[...]) + b2_ref[...], 0.0)
    m = jnp.mean(h2, axis=0, keepdims=True)
    z = jnp.maximum(
        jnp.dot(m, wc1_ref[...], preferred_element_type=jnp.float32)
        + bc1_ref[...], 0.0)
    o = jnp.dot(z, wc2_ref[...], preferred_element_type=jnp.float32) \
        + bc2_ref[...]
    out_ref[...] = jax.nn.sigmoid(o)


def kernel(x, edge_index, W1, b1, W2, b2, Wc1, bc1, Wc2, bc2):
    ei = edge_index.astype(jnp.int32)
    pad = EPAD - E
    src = jnp.concatenate([ei[0], jnp.zeros((pad,), jnp.int32)])
    dst = jnp.concatenate([ei[1], jnp.full((pad,), N, jnp.int32)])
    src = src.reshape(NW, CHUNKS, C)
    dst = dst.reshape(NW, CHUNKS, C)

    zeros_deg = jnp.zeros((TR, 1), jnp.float32)
    ones_col = jnp.ones((C, 1), jnp.float32)
    zeros_agg = jnp.zeros((TR, D_HID), jnp.float32)

    degp = _sc_deg(dst, zeros_deg, ones_col)

    g1, dis = pl.pallas_call(
        _tc_pre_body,
        out_shape=[
            jax.ShapeDtypeStruct((N, D_HID), jnp.float32),
            jax.ShapeDtypeStruct((N, 1), jnp.float32),
        ],
    )(x, W1, degp)

    acc1 = _sc_agg(g1, src, dst, zeros_agg)

    g2 = pl.pallas_call(
        _tc_mid_body,
        out_shape=jax.ShapeDtypeStruct((N, D_HID), jnp.float32),
    )(acc1, g1, dis, b1.reshape(1, D_HID), W2)

    acc2 = _sc_agg(g2, src, dst, zeros_agg)

    out = pl.pallas_call(
        _tc_fin_body,
        out_shape=jax.ShapeDtypeStruct((1, 1), jnp.float32),
    )(acc2, g2, dis, b2.reshape(1, D_HID), Wc1, bc1.reshape(1, D_HID // 2),
      Wc2, bc2.reshape(1, 1))
    return out
